# baseline (speedup 1.0000x reference)
import functools

import jax
import jax.numpy as jnp
from jax import lax
from jax.experimental import pallas as pl
from jax.experimental.pallas import tpu as pltpu
from jax.experimental.pallas import tpu_sc as plsc

D_MODEL = 64
BLK = 128
NUM_WORKERS = 32


def _make_kernel(batch, seq):
    assert batch == BLK * NUM_WORKERS
    mesh = plsc.VectorSubcoreMesh(core_axis_name="c", subcore_axis_name="s")

    @functools.partial(
        pl.kernel,
        mesh=mesh,
        out_type=jax.ShapeDtypeStruct(
            (seq, D_MODEL // 8, NUM_WORKERS, 8 * BLK), jnp.float32
        ),
        scratch_types=[
            pltpu.VMEM((BLK, seq), jnp.int32),
            pltpu.VMEM((seq, BLK), jnp.int32),
            [pltpu.VMEM((BLK, D_MODEL), jnp.float32) for _ in range(2)],
            [pltpu.VMEM((D_MODEL // 8, 8 * BLK), jnp.float32) for _ in range(2)],
            [pltpu.SemaphoreType.DMA for _ in range(2)],
            [pltpu.SemaphoreType.DMA for _ in range(2)],
        ],
        compiler_params=pltpu.CompilerParams(
            use_tc_tiling_on_sc=False, needs_layout_passes=False
        ),
    )
    def gather_kernel(tok_hbm, table_hbm, y_hbm, idx_raw, idx_t, rows, yb, sg, so):
        wid = lax.axis_index("s") * 2 + lax.axis_index("c")
        pltpu.sync_copy(tok_hbm.at[pl.ds(wid * BLK, BLK), :], idx_raw)

        iota = lax.iota(jnp.int32, 16)
        lanes = [iota + (16 * g) for g in range(BLK // 16)]

        def tbody(s, carry):
            col = jnp.zeros((16,), jnp.int32) + s
            for g in range(BLK // 16):
                v = plsc.load_gather(idx_raw, [lanes[g], col])
                idx_t[s, pl.ds(16 * g, 16)] = v
            return carry

        lax.fori_loop(0, seq, tbody, 0)

        def issue_gather(s, b):
            pltpu.async_copy(table_hbm.at[idx_t.at[s]], rows[b], sg[b])

        def wait_gather(b):
            pltpu.make_async_copy(
                table_hbm.at[pl.ds(0, BLK)], rows[b], sg[b]
            ).wait()

        def transpose(b):
            for d in range(D_MODEL):
                col = jnp.zeros((16,), jnp.int32) + d
                for g in range(BLK // 16):
                    v = plsc.load_gather(rows[b], [lanes[g], col])
                    yb[b][d // 8, pl.ds((d % 8) * BLK + 16 * g, 16)] = v

        def issue_store(s, b):
            pltpu.async_copy(yb[b], y_hbm.at[s, :, wid, :], so[b])

        def wait_store(b):
            pltpu.make_async_copy(
                yb[b], y_hbm.at[0, :, wid, :], so[b]
            ).wait()

        issue_gather(0, 0)
        issue_gather(1, 1)
        wait_gather(0)
        transpose(0)
        issue_gather(2, 0)
        issue_store(0, 0)
        wait_gather(1)
        transpose(1)
        issue_gather(3, 1)
        issue_store(1, 1)

        def body(i, carry):
            s0 = 2 * i
            wait_gather(0)
            wait_store(0)
            transpose(0)
            pl.when(i < seq // 2 - 1)(lambda: issue_gather(s0 + 2, 0))
            issue_store(s0, 0)
            wait_gather(1)
            wait_store(1)
            transpose(1)
            pl.when(i < seq // 2 - 1)(lambda: issue_gather(s0 + 3, 1))
            issue_store(s0 + 1, 1)
            return carry

        lax.fori_loop(1, seq // 2, body, 0)
        wait_store(0)
        wait_store(1)

    return gather_kernel


def kernel(tokens, token_emb):
    batch, seq = tokens.shape
    tok = tokens.astype(jnp.int32)
    y4 = _make_kernel(batch, seq)(tok, token_emb)
    y5 = y4.reshape(seq, D_MODEL // 8, NUM_WORKERS, 8, BLK)
    return y5.transpose(2, 4, 0, 1, 3).reshape(batch, seq, D_MODEL)


# speedup vs baseline: 1.4636x; 1.4636x over previous
import functools

import jax
import jax.numpy as jnp
from jax import lax
from jax.experimental import pallas as pl
from jax.experimental.pallas import tpu as pltpu
from jax.experimental.pallas import tpu_sc as plsc

D_MODEL = 64
BLK = 128
NUM_WORKERS = 32


def _make_kernel(batch, seq):
    assert batch == BLK * NUM_WORKERS
    mesh = plsc.VectorSubcoreMesh(core_axis_name="c", subcore_axis_name="s")

    @functools.partial(
        pl.kernel,
        mesh=mesh,
        out_type=jax.ShapeDtypeStruct(
            (seq, D_MODEL // 8, NUM_WORKERS, 8 * BLK), jnp.float32
        ),
        scratch_types=[
            pltpu.VMEM((BLK, seq), jnp.int32),
            pltpu.VMEM((seq, BLK), jnp.int32),
            [pltpu.VMEM((BLK, D_MODEL), jnp.float32) for _ in range(2)],
            [pltpu.VMEM((D_MODEL // 8, 8 * BLK), jnp.float32) for _ in range(2)],
            [pltpu.SemaphoreType.DMA for _ in range(2)],
            [pltpu.SemaphoreType.DMA for _ in range(2)],
        ],
        compiler_params=pltpu.CompilerParams(
            use_tc_tiling_on_sc=False, needs_layout_passes=False
        ),
    )
    def gather_kernel(tok_hbm, table_hbm, y_hbm, idx_raw, idx_t, rows, yb, sg, so):
        wid = lax.axis_index("s") * 2 + lax.axis_index("c")
        pltpu.sync_copy(tok_hbm.at[pl.ds(wid * BLK, BLK), :], idx_raw)

        iota = lax.iota(jnp.int32, 16)
        lanes = [iota + (16 * g) for g in range(BLK // 16)]

        @plsc.parallel_loop(0, seq, unroll=4)
        def _idx_transpose(s):
            col = jnp.zeros((16,), jnp.int32) + s
            for g in range(BLK // 16):
                v = plsc.load_gather(idx_raw, [lanes[g], col])
                idx_t[s, pl.ds(16 * g, 16)] = v

        dchunks = []
        for d0 in range(0, D_MODEL, 16):
            dv = iota + d0
            rvec = lax.shift_right_logical(dv, 3)
            cbase = lax.shift_left(jnp.bitwise_and(dv, 7), 7)
            dchunks.append((d0, rvec, cbase))

        def issue_gather(s, b):
            pltpu.async_copy(table_hbm.at[idx_t.at[s]], rows[b], sg[b])

        def wait_gather(b):
            pltpu.make_async_copy(
                table_hbm.at[pl.ds(0, BLK)], rows[b], sg[b]
            ).wait()

        def transpose(b):
            @plsc.parallel_loop(0, BLK, unroll=8)
            def _transpose(t):
                for d0, rvec, cbase in dchunks:
                    v = rows[b][t, pl.ds(d0, 16)]
                    plsc.store_scatter(yb[b], [rvec, cbase + t], v)

        def issue_store(s, b):
            pltpu.async_copy(yb[b], y_hbm.at[s, :, wid, :], so[b])

        def wait_store(b):
            pltpu.make_async_copy(
                yb[b], y_hbm.at[0, :, wid, :], so[b]
            ).wait()

        issue_gather(0, 0)
        issue_gather(1, 1)
        wait_gather(0)
        transpose(0)
        issue_gather(2, 0)
        issue_store(0, 0)
        wait_gather(1)
        transpose(1)
        issue_gather(3, 1)
        issue_store(1, 1)

        def body(i, carry):
            s0 = 2 * i
            wait_gather(0)
            wait_store(0)
            transpose(0)
            pl.when(i < seq // 2 - 1)(lambda: issue_gather(s0 + 2, 0))
            issue_store(s0, 0)
            wait_gather(1)
            wait_store(1)
            transpose(1)
            pl.when(i < seq // 2 - 1)(lambda: issue_gather(s0 + 3, 1))
            issue_store(s0 + 1, 1)
            return carry

        lax.fori_loop(1, seq // 2, body, 0)
        wait_store(0)
        wait_store(1)

    return gather_kernel


def kernel(tokens, token_emb):
    batch, seq = tokens.shape
    tok = tokens.astype(jnp.int32)
    y4 = _make_kernel(batch, seq)(tok, token_emb)
    y5 = y4.reshape(seq, D_MODEL // 8, NUM_WORKERS, 8, BLK)
    return y5.transpose(2, 4, 0, 1, 3).reshape(batch, seq, D_MODEL)


# speedup vs baseline: 2.3727x; 1.6211x over previous
import functools

import jax
import jax.numpy as jnp
from jax import lax
from jax.experimental import pallas as pl
from jax.experimental.pallas import tpu as pltpu
from jax.experimental.pallas import tpu_sc as plsc

D_MODEL = 64
BLK = 128
NUM_WORKERS = 32


def _make_kernel(batch, seq):
    assert batch == BLK * NUM_WORKERS
    mesh = plsc.VectorSubcoreMesh(core_axis_name="c", subcore_axis_name="s")

    @functools.partial(
        pl.kernel,
        mesh=mesh,
        out_type=jax.ShapeDtypeStruct(
            (seq, D_MODEL // 8, NUM_WORKERS, 8 * BLK), jnp.float32
        ),
        scratch_types=[
            pltpu.VMEM((BLK, seq + 1), jnp.int32),
            pltpu.VMEM((seq, BLK), jnp.int32),
            pltpu.VMEM((BLK, D_MODEL + 1), jnp.float32),
            [pltpu.VMEM((BLK, D_MODEL), jnp.float32) for _ in range(2)],
            [pltpu.VMEM((D_MODEL // 8, 8 * BLK), jnp.float32) for _ in range(2)],
            [pltpu.SemaphoreType.DMA for _ in range(2)],
            [pltpu.SemaphoreType.DMA for _ in range(2)],
        ],
        compiler_params=pltpu.CompilerParams(
            use_tc_tiling_on_sc=False, needs_layout_passes=False
        ),
    )
    def gather_kernel(
        tok_hbm, table_hbm, y_hbm, idx_raw, idx_t, rows_p, rows, yb, sg, so
    ):
        wid = lax.axis_index("s") * 2 + lax.axis_index("c")
        pltpu.sync_copy(
            tok_hbm.at[pl.ds(wid * BLK, BLK), :], idx_raw.at[:, pl.ds(0, seq)]
        )

        iota = lax.iota(jnp.int32, 16)
        lanes = [iota + (16 * g) for g in range(BLK // 16)]

        @plsc.parallel_loop(0, seq, unroll=4)
        def _idx_transpose(s):
            col = jnp.zeros((16,), jnp.int32) + s
            for g in range(BLK // 16):
                v = plsc.load_gather(idx_raw, [lanes[g], col])
                idx_t[s, pl.ds(16 * g, 16)] = v


        def issue_gather(s, b):
            pltpu.async_copy(table_hbm.at[idx_t.at[s]], rows[b], sg[b])

        def wait_gather(b):
            pltpu.make_async_copy(
                table_hbm.at[pl.ds(0, BLK)], rows[b], sg[b]
            ).wait()

        def transpose(b):
            @plsc.parallel_loop(0, BLK, unroll=8)
            def _repitch(t):
                for d0 in range(0, D_MODEL, 16):
                    rows_p[t, pl.ds(d0, 16)] = rows[b][t, pl.ds(d0, 16)]

            @plsc.parallel_loop(0, D_MODEL, unroll=4)
            def _transpose(d):
                col = jnp.zeros((16,), jnp.int32) + d
                row = lax.shift_right_logical(d, 3)
                off = lax.shift_left(jnp.bitwise_and(d, 7), 7)
                for g in range(BLK // 16):
                    v = plsc.load_gather(rows_p, [lanes[g], col])
                    yb[b][row, pl.ds(off + 16 * g, 16)] = v

        def issue_store(s, b):
            pltpu.async_copy(yb[b], y_hbm.at[s, :, wid, :], so[b])

        def wait_store(b):
            pltpu.make_async_copy(
                yb[b], y_hbm.at[0, :, wid, :], so[b]
            ).wait()

        issue_gather(0, 0)
        issue_gather(1, 1)
        wait_gather(0)
        transpose(0)
        issue_gather(2, 0)
        issue_store(0, 0)
        wait_gather(1)
        transpose(1)
        issue_gather(3, 1)
        issue_store(1, 1)

        def body(i, carry):
            s0 = 2 * i
            wait_gather(0)
            wait_store(0)
            transpose(0)
            pl.when(i < seq // 2 - 1)(lambda: issue_gather(s0 + 2, 0))
            issue_store(s0, 0)
            wait_gather(1)
            wait_store(1)
            transpose(1)
            pl.when(i < seq // 2 - 1)(lambda: issue_gather(s0 + 3, 1))
            issue_store(s0 + 1, 1)
            return carry

        lax.fori_loop(1, seq // 2, body, 0)
        wait_store(0)
        wait_store(1)

    return gather_kernel


def kernel(tokens, token_emb):
    batch, seq = tokens.shape
    tok = tokens.astype(jnp.int32)
    y4 = _make_kernel(batch, seq)(tok, token_emb)
    y5 = y4.reshape(seq, D_MODEL // 8, NUM_WORKERS, 8, BLK)
    return y5.transpose(2, 4, 0, 1, 3).reshape(batch, seq, D_MODEL)
